# Initial kernel scaffold; baseline (speedup 1.0000x reference)
import jax
import jax.numpy as jnp
from jax.experimental import pallas as pl


def kernel(features, rois):
    raise NotImplementedError("write your pallas kernel here")


# speedup vs baseline: 5.8369x; 5.8369x over previous
import functools
import numpy as np
import jax
import jax.numpy as jnp
from jax import lax
from jax.experimental import pallas as pl
from jax.experimental.pallas import tpu as pltpu, tpu_sc as plsc

S = 7
SS = S * S
SP = 52
P = SP * 4
H = W = 128
C = 256
CB = C // 16
SCALE = 0.125
NPAD = 5056
RPW = NPAD // 32
L = 16


def _offset_tables():
    oy = np.zeros(P, np.int32)
    ox = np.zeros(P, np.int32)
    for p in range(P):
        s, k = p // 4, p % 4
        if s < SS:
            sy, sx = s // S, s % S
            oy[p] = sy + 16 * (k // 2)
            ox[p] = sx + 16 * (k % 2)
    return oy, ox


_OY, _OX = _offset_tables()

_mesh = plsc.VectorSubcoreMesh(core_axis_name="c", subcore_axis_name="s")


@functools.partial(
    pl.kernel,
    mesh=_mesh,
    compiler_params=pltpu.CompilerParams(needs_layout_passes=False),
    out_type=jax.ShapeDtypeStruct((NPAD, C * SS), jnp.float32),
    scratch_types=[
        pltpu.VMEM((RPW * 4 + L, ), jnp.float32),
        pltpu.VMEM((P,), jnp.int32),
        pltpu.VMEM((P,), jnp.int32),
        pltpu.VMEM((2, 104), jnp.int32),
        pltpu.VMEM((P,), jnp.float32),
        pltpu.VMEM((32,), jnp.int32),
        pltpu.VMEM((32,), jnp.int32),
        pltpu.VMEM((32,), jnp.float32),
        pltpu.VMEM((32,), jnp.float32),
        pltpu.VMEM((P, C), jnp.float32),
        pltpu.VMEM((C * SS,), jnp.float32),
        pltpu.SemaphoreType.DMA,
    ],
)
def _roialign_sc(table, rois_p, oy_hbm, ox_hbm, out_hbm,
                 rois_v, oy_v, ox_v, idx2, wv, ybuf, xbuf, wyb, wxb,
                 rows, out_t, sem):
    wid = lax.axis_index("s") * 2 + lax.axis_index("c")
    base = wid * RPW
    pltpu.sync_copy(rois_p.at[pl.ds(base * 4, RPW * 4)], rois_v.at[pl.ds(0, RPW * 4)])
    pltpu.sync_copy(oy_hbm, oy_v)
    pltpu.sync_copy(ox_hbm, ox_v)

    iota = lax.broadcasted_iota(jnp.int32, (L,), 0)
    tvec = iota.astype(jnp.float32) * (1.0 / (S - 1))
    iota49 = iota * SS

    def roi_body(i, carry):
        rv = rois_v[pl.ds(4 * i, L)]
        x1 = jnp.clip(rv[0] * SCALE, 0.0, W - 1.0)
        y1 = jnp.clip(rv[1] * SCALE, 0.0, H - 1.0)
        x2 = jnp.clip(rv[2] * SCALE, 0.0, W - 1.0)
        y2 = jnp.clip(rv[3] * SCALE, 0.0, H - 1.0)
        xs = x1 + (x2 - x1) * tvec
        ys = y1 + (y2 - y1) * tvec
        x0r = xs.astype(jnp.int32)
        y0r = ys.astype(jnp.int32)
        fx = xs - x0r.astype(jnp.float32)
        fy = ys - y0r.astype(jnp.float32)
        x0c = jnp.clip(x0r, 0, W - 1)
        x1c = jnp.minimum(x0c + 1, W - 1)
        y0c = jnp.clip(y0r, 0, H - 1)
        y1c = jnp.minimum(y0c + 1, H - 1)
        ybuf[pl.ds(0, L)] = y0c * W
        ybuf[pl.ds(L, L)] = y1c * W
        xbuf[pl.ds(0, L)] = x0c
        xbuf[pl.ds(L, L)] = x1c
        wyb[pl.ds(0, L)] = 1.0 - fy
        wyb[pl.ds(L, L)] = fy
        wxb[pl.ds(0, L)] = 1.0 - fx
        wxb[pl.ds(L, L)] = fx

        for c in range(P // L):
            oyc = oy_v[pl.ds(c * L, L)]
            oxc = ox_v[pl.ds(c * L, L)]
            yg = plsc.load_gather(ybuf, [oyc])
            xg = plsc.load_gather(xbuf, [oxc])
            pv = iota + (c * L)
            gv = (pv >= 104).astype(jnp.int32)
            jv = pv - gv * 104
            plsc.store_scatter(idx2, [gv, jv], yg + xg)
            wyv = plsc.load_gather(wyb, [oyc])
            wxv = plsc.load_gather(wxb, [oxc])
            wv[pl.ds(c * L, L)] = wyv * wxv

        cp0 = pltpu.async_copy(table.at[idx2.at[0]], rows.at[pl.ds(0, 104)], sem)
        cp1 = pltpu.async_copy(table.at[idx2.at[1]], rows.at[pl.ds(104, 104)], sem)
        cp0.wait()
        cp1.wait()

        def s_body(s, carry2):
            b = 4 * s
            wvv = wv[pl.ds(b, L)]
            w00 = wvv[0]
            w01 = wvv[1]
            w10 = wvv[2]
            w11 = wvv[3]
            for cb in range(CB):
                sl = pl.ds(cb * L, L)
                acc = (rows[b, sl] * w00 + rows[b + 1, sl] * w01
                       + rows[b + 2, sl] * w10 + rows[b + 3, sl] * w11)
                plsc.store_scatter(out_t, [iota49 + (cb * L * SS) + s], acc)
            return carry2

        lax.fori_loop(0, SS, s_body, 0)
        pltpu.sync_copy(out_t, out_hbm.at[base + i])
        return carry

    lax.fori_loop(0, RPW, roi_body, 0)


def kernel(features, rois):
    feat = features[0]
    table = jnp.transpose(feat, (1, 2, 0)).reshape(H * W, C)
    n = rois.shape[0]
    rois_p = jnp.zeros((NPAD * 4,), jnp.float32).at[:n * 4].set(rois.reshape(-1))
    oy = jnp.asarray(_OY)
    ox = jnp.asarray(_OX)
    out = _roialign_sc(table, rois_p, oy, ox)
    return out[:n].reshape(n, C, S, S)


# speedup vs baseline: 7.8198x; 1.3397x over previous
import functools
import numpy as np
import jax
import jax.numpy as jnp
from jax import lax
from jax.experimental import pallas as pl
from jax.experimental.pallas import tpu as pltpu, tpu_sc as plsc

S = 7
SS = S * S
G = 200
P = 208
H = W = 128
C = 256
CB = C // 16
SCALE = 0.125
NPAD = 5056
RPW = NPAD // 32
L = 16


def _offset_tables():
    oy = np.zeros(P, np.int32)
    ox = np.zeros(P, np.int32)
    for p in range(P):
        s, k = p // 4, p % 4
        if s < SS:
            sy, sx = s // S, s % S
            oy[p] = sy + 16 * (k // 2)
            ox[p] = sx + 16 * (k % 2)
    return oy, ox


_OY, _OX = _offset_tables()

_mesh = plsc.VectorSubcoreMesh(core_axis_name="c", subcore_axis_name="s")


@functools.partial(
    pl.kernel,
    mesh=_mesh,
    compiler_params=pltpu.CompilerParams(needs_layout_passes=False),
    out_type=jax.ShapeDtypeStruct((NPAD, C * SS), jnp.float32),
    scratch_types=[
        pltpu.VMEM((RPW * 4 + L,), jnp.float32),
        pltpu.VMEM((P,), jnp.int32),
        pltpu.VMEM((P,), jnp.int32),
        pltpu.VMEM((104,), jnp.int32),
        pltpu.VMEM((104,), jnp.int32),
        pltpu.VMEM((96,), jnp.int32),
        pltpu.VMEM((96,), jnp.int32),
        pltpu.VMEM((P,), jnp.float32),
        pltpu.VMEM((P,), jnp.float32),
        pltpu.VMEM((32,), jnp.int32),
        pltpu.VMEM((32,), jnp.int32),
        pltpu.VMEM((32,), jnp.float32),
        pltpu.VMEM((32,), jnp.float32),
        pltpu.VMEM((G, C), jnp.float32),
        pltpu.VMEM((G, C), jnp.float32),
        pltpu.VMEM((C * SS,), jnp.float32),
        pltpu.VMEM((C * SS,), jnp.float32),
        pltpu.SemaphoreType.DMA,
        pltpu.SemaphoreType.DMA,
        pltpu.SemaphoreType.DMA,
        pltpu.SemaphoreType.DMA,
    ],
)
def _roialign_sc(table, rois_p, oy_hbm, ox_hbm, out_hbm,
                 rois_v, oy_v, ox_v, idxa0, idxa1, idxb0, idxb1, wv0, wv1,
                 ybuf, xbuf, wyb, wxb, rows0, rows1, out_t0, out_t1,
                 gsemA, gsemB, osemA, osemB):
    idxa = [idxa0, idxa1]
    idxb = [idxb0, idxb1]
    wv = [wv0, wv1]
    rows = [rows0, rows1]
    out_t = [out_t0, out_t1]
    wid = lax.axis_index("s") * 2 + lax.axis_index("c")
    base = wid * RPW
    pltpu.sync_copy(rois_p.at[pl.ds(base * 4, RPW * 4)], rois_v.at[pl.ds(0, RPW * 4)])
    pltpu.sync_copy(oy_hbm, oy_v)
    pltpu.sync_copy(ox_hbm, ox_v)

    iota = lax.broadcasted_iota(jnp.int32, (L,), 0)
    tvec = iota.astype(jnp.float32) * (1.0 / (S - 1))
    sidx_base = [iota * SS + cb * (L * SS) for cb in range(CB)]

    def setup(i, slot):
        rv = rois_v[pl.ds(4 * i, L)]
        x1 = jnp.clip(rv[0] * SCALE, 0.0, W - 1.0)
        y1 = jnp.clip(rv[1] * SCALE, 0.0, H - 1.0)
        x2 = jnp.clip(rv[2] * SCALE, 0.0, W - 1.0)
        y2 = jnp.clip(rv[3] * SCALE, 0.0, H - 1.0)
        xs = x1 + (x2 - x1) * tvec
        ys = y1 + (y2 - y1) * tvec
        x0r = xs.astype(jnp.int32)
        y0r = ys.astype(jnp.int32)
        fx = xs - x0r.astype(jnp.float32)
        fy = ys - y0r.astype(jnp.float32)
        x0c = jnp.clip(x0r, 0, W - 1)
        x1c = jnp.minimum(x0c + 1, W - 1)
        y0c = jnp.clip(y0r, 0, H - 1)
        y1c = jnp.minimum(y0c + 1, H - 1)
        ybuf[pl.ds(0, L)] = y0c * W
        ybuf[pl.ds(L, L)] = y1c * W
        xbuf[pl.ds(0, L)] = x0c
        xbuf[pl.ds(L, L)] = x1c
        wyb[pl.ds(0, L)] = 1.0 - fy
        wyb[pl.ds(L, L)] = fy
        wxb[pl.ds(0, L)] = 1.0 - fx
        wxb[pl.ds(L, L)] = fx
        for c in range(P // L):
            oyc = oy_v[pl.ds(c * L, L)]
            oxc = ox_v[pl.ds(c * L, L)]
            yg = plsc.load_gather(ybuf, [oyc])
            xg = plsc.load_gather(xbuf, [oxc])
            pv = iota + (c * L)
            vals = yg + xg
            plsc.store_scatter(idxa[slot], [jnp.minimum(pv, 103)], vals,
                               mask=pv < 104)
            plsc.store_scatter(idxb[slot], [jnp.clip(pv - 104, 0, 95)], vals,
                               mask=jnp.logical_and(pv >= 104, pv < G))
            wyv = plsc.load_gather(wyb, [oyc])
            wxv = plsc.load_gather(wxb, [oxc])
            wv[slot][pl.ds(c * L, L)] = wyv * wxv

    def fire_gather(slot, gsem):
        pltpu.async_copy(table.at[idxa[slot]],
                         rows[slot].at[pl.ds(0, 104)], gsem)
        pltpu.async_copy(table.at[idxb[slot]],
                         rows[slot].at[pl.ds(104, 96)], gsem)

    def wait_gather(slot, gsem):
        pltpu.make_async_copy(table.at[idxa[slot]],
                              rows[slot].at[pl.ds(0, 104)], gsem).wait()
        pltpu.make_async_copy(table.at[idxb[slot]],
                              rows[slot].at[pl.ds(104, 96)], gsem).wait()

    def combine(i, slot):
        def s_body(s, carry):
            b = 4 * s
            wvv = wv[slot][pl.ds(b, L)]
            w00 = wvv[0]
            w01 = wvv[1]
            w10 = wvv[2]
            w11 = wvv[3]
            for cb in range(CB):
                sl = pl.ds(cb * L, L)
                r = rows[slot]
                acc = (r[b, sl] * w00 + r[b + 1, sl] * w01
                       + r[b + 2, sl] * w10 + r[b + 3, sl] * w11)
                plsc.store_scatter(out_t[slot], [sidx_base[cb] + s], acc)
            return carry
        lax.fori_loop(0, SS, s_body, 0)

    def fire_out(i, slot, osem):
        pltpu.async_copy(out_t[slot], out_hbm.at[base + i], osem)

    def wait_out(slot, osem):
        pltpu.make_async_copy(out_t[slot], out_hbm.at[base], osem).wait()

    setup(0, 0)
    fire_gather(0, gsemA)
    setup(1, 1)
    fire_gather(1, gsemB)

    def body(j, carry):
        i0 = 2 * j
        wait_gather(0, gsemA)

        @pl.when(j > 0)
        def _():
            wait_out(0, osemA)
        combine(i0, 0)
        fire_out(i0, 0, osemA)

        @pl.when(i0 + 2 < RPW)
        def _():
            setup(i0 + 2, 0)
            fire_gather(0, gsemA)

        wait_gather(1, gsemB)

        @pl.when(j > 0)
        def _():
            wait_out(1, osemB)
        combine(i0 + 1, 1)
        fire_out(i0 + 1, 1, osemB)

        @pl.when(i0 + 3 < RPW)
        def _():
            setup(i0 + 3, 1)
            fire_gather(1, gsemB)
        return carry

    lax.fori_loop(0, RPW // 2, body, 0)
    wait_out(0, osemA)
    wait_out(1, osemB)


def kernel(features, rois):
    feat = features[0]
    table = jnp.transpose(feat, (1, 2, 0)).reshape(H * W, C)
    n = rois.shape[0]
    rois_p = jnp.zeros((NPAD * 4,), jnp.float32).at[:n * 4].set(rois.reshape(-1))
    oy = jnp.asarray(_OY)
    ox = jnp.asarray(_OX)
    out = _roialign_sc(table, rois_p, oy, ox)
    return out[:n].reshape(n, C, S, S)


# speedup vs baseline: 8.4597x; 1.0818x over previous
import functools
import numpy as np
import jax
import jax.numpy as jnp
from jax import lax
from jax.experimental import pallas as pl
from jax.experimental.pallas import tpu as pltpu, tpu_sc as plsc

S = 7
SS = S * S
G = 200
P = 208
H = W = 128
C = 256
CB = C // 16
SCALE = 0.125
N = 5000
RPW = 2 * ((N + 63) // 64)
L = 16


def _offset_tables():
    oy = np.zeros(P, np.int32)
    ox = np.zeros(P, np.int32)
    for p in range(P):
        s, k = p // 4, p % 4
        if s < SS:
            sy, sx = s // S, s % S
            oy[p] = sy + 16 * (k // 2)
            ox[p] = sx + 16 * (k % 2)
    return oy, ox


_OY, _OX = _offset_tables()

_mesh = plsc.VectorSubcoreMesh(core_axis_name="c", subcore_axis_name="s")


@functools.partial(
    pl.kernel,
    mesh=_mesh,
    compiler_params=pltpu.CompilerParams(needs_layout_passes=False),
    out_type=jax.ShapeDtypeStruct((N, C * SS), jnp.float32),
    scratch_types=[
        pltpu.VMEM((RPW * 4 + L,), jnp.float32),
        pltpu.VMEM((P,), jnp.int32),
        pltpu.VMEM((P,), jnp.int32),
        pltpu.VMEM((104,), jnp.int32),
        pltpu.VMEM((104,), jnp.int32),
        pltpu.VMEM((96,), jnp.int32),
        pltpu.VMEM((96,), jnp.int32),
        pltpu.VMEM((P,), jnp.float32),
        pltpu.VMEM((P,), jnp.float32),
        pltpu.VMEM((32,), jnp.int32),
        pltpu.VMEM((32,), jnp.int32),
        pltpu.VMEM((32,), jnp.float32),
        pltpu.VMEM((32,), jnp.float32),
        pltpu.VMEM((G, C), jnp.float32),
        pltpu.VMEM((G, C), jnp.float32),
        pltpu.VMEM((C * SS,), jnp.float32),
        pltpu.VMEM((C * SS,), jnp.float32),
        pltpu.SemaphoreType.DMA,
        pltpu.SemaphoreType.DMA,
        pltpu.SemaphoreType.DMA,
        pltpu.SemaphoreType.DMA,
    ],
)
def _roialign_sc(table, rois_p, oy_hbm, ox_hbm, out_hbm,
                 rois_v, oy_v, ox_v, idxa0, idxa1, idxb0, idxb1, wv0, wv1,
                 ybuf, xbuf, wyb, wxb, rows0, rows1, out_t0, out_t1,
                 gsemA, gsemB, osemA, osemB):
    idxa = [idxa0, idxa1]
    idxb = [idxb0, idxb1]
    wv = [wv0, wv1]
    rows = [rows0, rows1]
    out_t = [out_t0, out_t1]
    wid = lax.axis_index("s") * 2 + lax.axis_index("c")
    base = 2 * ((wid * N) // 64)
    cnt = 2 * (((wid + 1) * N) // 64) - base
    pltpu.sync_copy(rois_p.at[pl.ds(base * 4, RPW * 4)], rois_v.at[pl.ds(0, RPW * 4)])
    pltpu.sync_copy(oy_hbm, oy_v)
    pltpu.sync_copy(ox_hbm, ox_v)

    iota = lax.broadcasted_iota(jnp.int32, (L,), 0)
    tvec = iota.astype(jnp.float32) * (1.0 / (S - 1))
    sidx_base = [iota * SS + cb * (L * SS) for cb in range(CB)]

    def setup(i, slot):
        rv = rois_v[pl.ds(4 * i, L)]
        x1 = jnp.clip(rv[0] * SCALE, 0.0, W - 1.0)
        y1 = jnp.clip(rv[1] * SCALE, 0.0, H - 1.0)
        x2 = jnp.clip(rv[2] * SCALE, 0.0, W - 1.0)
        y2 = jnp.clip(rv[3] * SCALE, 0.0, H - 1.0)
        xs = x1 + (x2 - x1) * tvec
        ys = y1 + (y2 - y1) * tvec
        x0r = xs.astype(jnp.int32)
        y0r = ys.astype(jnp.int32)
        fx = xs - x0r.astype(jnp.float32)
        fy = ys - y0r.astype(jnp.float32)
        x0c = jnp.clip(x0r, 0, W - 1)
        x1c = jnp.minimum(x0c + 1, W - 1)
        y0c = jnp.clip(y0r, 0, H - 1)
        y1c = jnp.minimum(y0c + 1, H - 1)
        ybuf[pl.ds(0, L)] = y0c * W
        ybuf[pl.ds(L, L)] = y1c * W
        xbuf[pl.ds(0, L)] = x0c
        xbuf[pl.ds(L, L)] = x1c
        wyb[pl.ds(0, L)] = 1.0 - fy
        wyb[pl.ds(L, L)] = fy
        wxb[pl.ds(0, L)] = 1.0 - fx
        wxb[pl.ds(L, L)] = fx
        for c in range(P // L):
            oyc = oy_v[pl.ds(c * L, L)]
            oxc = ox_v[pl.ds(c * L, L)]
            yg = plsc.load_gather(ybuf, [oyc])
            xg = plsc.load_gather(xbuf, [oxc])
            pv = iota + (c * L)
            vals = yg + xg
            plsc.store_scatter(idxa[slot], [jnp.minimum(pv, 103)], vals,
                               mask=pv < 104)
            plsc.store_scatter(idxb[slot], [jnp.clip(pv - 104, 0, 95)], vals,
                               mask=jnp.logical_and(pv >= 104, pv < G))
            wyv = plsc.load_gather(wyb, [oyc])
            wxv = plsc.load_gather(wxb, [oxc])
            wv[slot][pl.ds(c * L, L)] = wyv * wxv

    def fire_gather(slot, gsem):
        pltpu.async_copy(table.at[idxa[slot]],
                         rows[slot].at[pl.ds(0, 104)], gsem)
        pltpu.async_copy(table.at[idxb[slot]],
                         rows[slot].at[pl.ds(104, 96)], gsem)

    def wait_gather(slot, gsem):
        pltpu.make_async_copy(table.at[idxa[slot]],
                              rows[slot].at[pl.ds(0, 104)], gsem).wait()
        pltpu.make_async_copy(table.at[idxb[slot]],
                              rows[slot].at[pl.ds(104, 96)], gsem).wait()

    def combine(i, slot):
        def s_body(s, carry):
            b = 4 * s
            wvv = wv[slot][pl.ds(b, L)]
            w00 = wvv[0]
            w01 = wvv[1]
            w10 = wvv[2]
            w11 = wvv[3]
            for cb in range(CB):
                sl = pl.ds(cb * L, L)
                r = rows[slot]
                acc = (r[b, sl] * w00 + r[b + 1, sl] * w01
                       + r[b + 2, sl] * w10 + r[b + 3, sl] * w11)
                plsc.store_scatter(out_t[slot], [sidx_base[cb] + s], acc)
            return carry
        lax.fori_loop(0, SS, s_body, 0)

    def fire_out(i, slot, osem):
        pltpu.async_copy(out_t[slot], out_hbm.at[base + i], osem)

    def wait_out(slot, osem):
        pltpu.make_async_copy(out_t[slot], out_hbm.at[base], osem).wait()

    setup(0, 0)
    fire_gather(0, gsemA)
    setup(1, 1)
    fire_gather(1, gsemB)

    def body(j, carry):
        i0 = 2 * j
        wait_gather(0, gsemA)

        @pl.when(j > 0)
        def _():
            wait_out(0, osemA)
        combine(i0, 0)
        fire_out(i0, 0, osemA)

        @pl.when(i0 + 2 < cnt)
        def _():
            setup(i0 + 2, 0)
            fire_gather(0, gsemA)

        wait_gather(1, gsemB)

        @pl.when(j > 0)
        def _():
            wait_out(1, osemB)
        combine(i0 + 1, 1)
        fire_out(i0 + 1, 1, osemB)

        @pl.when(i0 + 3 < cnt)
        def _():
            setup(i0 + 3, 1)
            fire_gather(1, gsemB)
        return carry

    lax.fori_loop(0, cnt // 2, body, 0)
    wait_out(0, osemA)
    wait_out(1, osemB)


def kernel(features, rois):
    feat = features[0]
    table = jnp.transpose(feat, (1, 2, 0)).reshape(H * W, C)
    n = rois.shape[0]
    oy = jnp.asarray(_OY)
    ox = jnp.asarray(_OX)
    out = _roialign_sc(table, rois.reshape(-1), oy, ox)
    return out.reshape(n, C, S, S)


# speedup vs baseline: 10.7315x; 1.2685x over previous
import functools
import numpy as np
import jax
import jax.numpy as jnp
from jax import lax
from jax.experimental import pallas as pl
from jax.experimental.pallas import tpu as pltpu, tpu_sc as plsc

S = 7
SS = S * S
G = 200
P = 208
H = W = 128
C = 256
CB = C // 16
SCALE = 0.125
N = 5000
RPW = 2 * ((N + 63) // 64)
L = 16


def _offset_tables():
    oy = np.zeros(P, np.int32)
    ox = np.zeros(P, np.int32)
    for p in range(P):
        s, k = p // 4, p % 4
        if s < SS:
            sy, sx = s // S, s % S
            oy[p] = sy + 16 * (k // 2)
            ox[p] = sx + 16 * (k % 2)
    return oy, ox


_OY, _OX = _offset_tables()

_mesh = plsc.VectorSubcoreMesh(core_axis_name="c", subcore_axis_name="s")


@functools.partial(
    pl.kernel,
    mesh=_mesh,
    compiler_params=pltpu.CompilerParams(needs_layout_passes=False),
    out_type=jax.ShapeDtypeStruct((N, C * SS), jnp.float32),
    scratch_types=[
        pltpu.VMEM((RPW * 4 + L,), jnp.float32),
        pltpu.VMEM((P,), jnp.int32),
        pltpu.VMEM((P,), jnp.int32),
        pltpu.VMEM((104,), jnp.int32),
        pltpu.VMEM((104,), jnp.int32),
        pltpu.VMEM((96,), jnp.int32),
        pltpu.VMEM((96,), jnp.int32),
        pltpu.VMEM((P,), jnp.float32),
        pltpu.VMEM((P,), jnp.float32),
        pltpu.VMEM((32,), jnp.int32),
        pltpu.VMEM((32,), jnp.int32),
        pltpu.VMEM((32,), jnp.float32),
        pltpu.VMEM((32,), jnp.float32),
        pltpu.VMEM((G, C // 2), jnp.int32),
        pltpu.VMEM((G, C // 2), jnp.int32),
        pltpu.VMEM((C * SS,), jnp.float32),
        pltpu.VMEM((C * SS,), jnp.float32),
        pltpu.SemaphoreType.DMA,
        pltpu.SemaphoreType.DMA,
        pltpu.SemaphoreType.DMA,
        pltpu.SemaphoreType.DMA,
    ],
)
def _roialign_sc(table, rois_p, oy_hbm, ox_hbm, out_hbm,
                 rois_v, oy_v, ox_v, idxa0, idxa1, idxb0, idxb1, wv0, wv1,
                 ybuf, xbuf, wyb, wxb, rows0, rows1, out_t0, out_t1,
                 gsemA, gsemB, osemA, osemB):
    idxa = [idxa0, idxa1]
    idxb = [idxb0, idxb1]
    wv = [wv0, wv1]
    rows = [rows0, rows1]
    out_t = [out_t0, out_t1]
    wid = lax.axis_index("s") * 2 + lax.axis_index("c")
    base = 2 * ((wid * N) // 64)
    cnt = 2 * (((wid + 1) * N) // 64) - base
    pltpu.sync_copy(rois_p.at[pl.ds(base * 4, RPW * 4)], rois_v.at[pl.ds(0, RPW * 4)])
    pltpu.sync_copy(oy_hbm, oy_v)
    pltpu.sync_copy(ox_hbm, ox_v)

    iota = lax.broadcasted_iota(jnp.int32, (L,), 0)
    tvec = iota.astype(jnp.float32) * (1.0 / (S - 1))
    sidx_base = [iota * (2 * SS) + cb * (2 * L * SS) for cb in range(CB // 2)]

    def setup(i, slot):
        rv = rois_v[pl.ds(4 * i, L)]
        x1 = jnp.clip(rv[0] * SCALE, 0.0, W - 1.0)
        y1 = jnp.clip(rv[1] * SCALE, 0.0, H - 1.0)
        x2 = jnp.clip(rv[2] * SCALE, 0.0, W - 1.0)
        y2 = jnp.clip(rv[3] * SCALE, 0.0, H - 1.0)
        xs = x1 + (x2 - x1) * tvec
        ys = y1 + (y2 - y1) * tvec
        x0r = xs.astype(jnp.int32)
        y0r = ys.astype(jnp.int32)
        fx = xs - x0r.astype(jnp.float32)
        fy = ys - y0r.astype(jnp.float32)
        x0c = jnp.clip(x0r, 0, W - 1)
        x1c = jnp.minimum(x0c + 1, W - 1)
        y0c = jnp.clip(y0r, 0, H - 1)
        y1c = jnp.minimum(y0c + 1, H - 1)
        ybuf[pl.ds(0, L)] = y0c * W
        ybuf[pl.ds(L, L)] = y1c * W
        xbuf[pl.ds(0, L)] = x0c
        xbuf[pl.ds(L, L)] = x1c
        wyb[pl.ds(0, L)] = 1.0 - fy
        wyb[pl.ds(L, L)] = fy
        wxb[pl.ds(0, L)] = 1.0 - fx
        wxb[pl.ds(L, L)] = fx
        for c in range(P // L):
            oyc = oy_v[pl.ds(c * L, L)]
            oxc = ox_v[pl.ds(c * L, L)]
            yg = plsc.load_gather(ybuf, [oyc])
            xg = plsc.load_gather(xbuf, [oxc])
            pv = iota + (c * L)
            vals = yg + xg
            plsc.store_scatter(idxa[slot], [jnp.minimum(pv, 103)], vals,
                               mask=pv < 104)
            plsc.store_scatter(idxb[slot], [jnp.clip(pv - 104, 0, 95)], vals,
                               mask=jnp.logical_and(pv >= 104, pv < G))
            wyv = plsc.load_gather(wyb, [oyc])
            wxv = plsc.load_gather(wxb, [oxc])
            wv[slot][pl.ds(c * L, L)] = wyv * wxv

    def fire_gather(slot, gsem):
        pltpu.async_copy(table.at[idxa[slot]],
                         rows[slot].at[pl.ds(0, 104)], gsem)
        pltpu.async_copy(table.at[idxb[slot]],
                         rows[slot].at[pl.ds(104, 96)], gsem)

    def wait_gather(slot, gsem):
        pltpu.make_async_copy(table.at[idxa[slot]],
                              rows[slot].at[pl.ds(0, 104)], gsem).wait()
        pltpu.make_async_copy(table.at[idxb[slot]],
                              rows[slot].at[pl.ds(104, 96)], gsem).wait()

    def combine(i, slot):
        def s_body(s, carry):
            b = 4 * s
            wvv = wv[slot][pl.ds(b, L)]
            ws = [wvv[0], wvv[1], wvv[2], wvv[3]]
            for cb in range(CB // 2):
                sl = pl.ds(cb * L, L)
                r = rows[slot]
                acc_e = None
                acc_o = None
                for k in range(4):
                    e, o = plsc.unpack(
                        plsc.bitcast(r[b + k, sl], jnp.bfloat16),
                        format=plsc.PackFormat.INTERLEAVED)
                    if acc_e is None:
                        acc_e = e * ws[k]
                        acc_o = o * ws[k]
                    else:
                        acc_e = acc_e + e * ws[k]
                        acc_o = acc_o + o * ws[k]
                plsc.store_scatter(out_t[slot], [sidx_base[cb] + s], acc_e)
                plsc.store_scatter(out_t[slot], [sidx_base[cb] + (SS + s)], acc_o)
            return carry
        lax.fori_loop(0, SS, s_body, 0)

    def fire_out(i, slot, osem):
        pltpu.async_copy(out_t[slot], out_hbm.at[base + i], osem)

    def wait_out(slot, osem):
        pltpu.make_async_copy(out_t[slot], out_hbm.at[base], osem).wait()

    setup(0, 0)
    fire_gather(0, gsemA)
    setup(1, 1)
    fire_gather(1, gsemB)

    def body(j, carry):
        i0 = 2 * j
        wait_gather(0, gsemA)

        @pl.when(j > 0)
        def _():
            wait_out(0, osemA)
        combine(i0, 0)
        fire_out(i0, 0, osemA)

        @pl.when(i0 + 2 < cnt)
        def _():
            setup(i0 + 2, 0)
            fire_gather(0, gsemA)

        wait_gather(1, gsemB)

        @pl.when(j > 0)
        def _():
            wait_out(1, osemB)
        combine(i0 + 1, 1)
        fire_out(i0 + 1, 1, osemB)

        @pl.when(i0 + 3 < cnt)
        def _():
            setup(i0 + 3, 1)
            fire_gather(1, gsemB)
        return carry

    lax.fori_loop(0, cnt // 2, body, 0)
    wait_out(0, osemA)
    wait_out(1, osemB)


def kernel(features, rois):
    feat = features[0]
    table = jnp.transpose(feat, (1, 2, 0)).reshape(H * W, C)
    table = jax.lax.bitcast_convert_type(
        table.astype(jnp.bfloat16).reshape(H * W, C // 2, 2), jnp.int32)
    n = rois.shape[0]
    oy = jnp.asarray(_OY)
    ox = jnp.asarray(_OX)
    out = _roialign_sc(table, rois.reshape(-1), oy, ox)
    return out.reshape(n, C, S, S)


# speedup vs baseline: 20.6010x; 1.9197x over previous
import functools
import numpy as np
import jax
import jax.numpy as jnp
from jax import lax
from jax.experimental import pallas as pl
from jax.experimental.pallas import tpu as pltpu, tpu_sc as plsc

S = 7
SS = S * S
G = 200
P = 208
H = W = 128
C = 256
CB = C // 16
SCALE = 0.125
N = 5000
RPW = 2 * ((N + 63) // 64)
L = 16


def _offset_tables():
    oy = np.zeros(P, np.int32)
    ox = np.zeros(P, np.int32)
    for p in range(P):
        s, k = p // 4, p % 4
        if s < SS:
            sy, sx = s // S, s % S
            oy[p] = sy + 16 * (k // 2)
            ox[p] = sx + 16 * (k % 2)
    return oy, ox


_OY, _OX = _offset_tables()


def _out_row_table():
    oz = np.zeros(112, np.int32)
    for r in range(98):
        s, ct = r // 2, r % 2
        oz[r] = s * 10000 + ct * 8
    return oz


_OZ = _out_row_table()

_mesh = plsc.VectorSubcoreMesh(core_axis_name="c", subcore_axis_name="s")


@functools.partial(
    pl.kernel,
    mesh=_mesh,
    compiler_params=pltpu.CompilerParams(needs_layout_passes=False),
    out_type=jax.ShapeDtypeStruct((N * C * SS // 128, 128), jnp.float32),
    scratch_types=[
        pltpu.VMEM((RPW * 4 + L,), jnp.float32),
        pltpu.VMEM((P,), jnp.int32),
        pltpu.VMEM((P,), jnp.int32),
        pltpu.VMEM((112,), jnp.int32),
        pltpu.VMEM((98,), jnp.int32),
        pltpu.VMEM((98,), jnp.int32),
        pltpu.VMEM((104,), jnp.int32),
        pltpu.VMEM((104,), jnp.int32),
        pltpu.VMEM((96,), jnp.int32),
        pltpu.VMEM((96,), jnp.int32),
        pltpu.VMEM((P,), jnp.float32),
        pltpu.VMEM((P,), jnp.float32),
        pltpu.VMEM((32,), jnp.int32),
        pltpu.VMEM((32,), jnp.int32),
        pltpu.VMEM((32,), jnp.float32),
        pltpu.VMEM((32,), jnp.float32),
        pltpu.VMEM((G, C // 2), jnp.int32),
        pltpu.VMEM((G, C // 2), jnp.int32),
        pltpu.VMEM((98, 128), jnp.float32),
        pltpu.VMEM((98, 128), jnp.float32),
        pltpu.SemaphoreType.DMA,
        pltpu.SemaphoreType.DMA,
        pltpu.SemaphoreType.DMA,
        pltpu.SemaphoreType.DMA,
    ],
)
def _roialign_sc(table, rois_p, oy_hbm, ox_hbm, oz_hbm, out_hbm,
                 rois_v, oy_v, ox_v, oz_v, idxo0, idxo1,
                 idxa0, idxa1, idxb0, idxb1, wv0, wv1,
                 ybuf, xbuf, wyb, wxb, rows0, rows1, out_t0, out_t1,
                 gsemA, gsemB, osemA, osemB):
    idxo = [idxo0, idxo1]
    idxa = [idxa0, idxa1]
    idxb = [idxb0, idxb1]
    wv = [wv0, wv1]
    rows = [rows0, rows1]
    out_t = [out_t0, out_t1]
    wid = lax.axis_index("s") * 2 + lax.axis_index("c")
    base = 2 * ((wid * N) // 64)
    cnt = 2 * (((wid + 1) * N) // 64) - base
    pltpu.sync_copy(rois_p.at[pl.ds(base * 4, RPW * 4)], rois_v.at[pl.ds(0, RPW * 4)])
    pltpu.sync_copy(oy_hbm, oy_v)
    pltpu.sync_copy(ox_hbm, ox_v)
    pltpu.sync_copy(oz_hbm, oz_v)

    iota = lax.broadcasted_iota(jnp.int32, (L,), 0)
    tvec = iota.astype(jnp.float32) * (1.0 / (S - 1))
    col_base = [iota * 2 + 32 * (cb % 4) for cb in range(CB // 2)]

    def setup(i, slot):
        rv = rois_v[pl.ds(4 * i, L)]
        x1 = jnp.clip(rv[0] * SCALE, 0.0, W - 1.0)
        y1 = jnp.clip(rv[1] * SCALE, 0.0, H - 1.0)
        x2 = jnp.clip(rv[2] * SCALE, 0.0, W - 1.0)
        y2 = jnp.clip(rv[3] * SCALE, 0.0, H - 1.0)
        xs = x1 + (x2 - x1) * tvec
        ys = y1 + (y2 - y1) * tvec
        x0r = xs.astype(jnp.int32)
        y0r = ys.astype(jnp.int32)
        fx = xs - x0r.astype(jnp.float32)
        fy = ys - y0r.astype(jnp.float32)
        x0c = jnp.clip(x0r, 0, W - 1)
        x1c = jnp.minimum(x0c + 1, W - 1)
        y0c = jnp.clip(y0r, 0, H - 1)
        y1c = jnp.minimum(y0c + 1, H - 1)
        ybuf[pl.ds(0, L)] = y0c * W
        ybuf[pl.ds(L, L)] = y1c * W
        xbuf[pl.ds(0, L)] = x0c
        xbuf[pl.ds(L, L)] = x1c
        wyb[pl.ds(0, L)] = 1.0 - fy
        wyb[pl.ds(L, L)] = fy
        wxb[pl.ds(0, L)] = 1.0 - fx
        wxb[pl.ds(L, L)] = fx
        for c in range(P // L):
            oyc = oy_v[pl.ds(c * L, L)]
            oxc = ox_v[pl.ds(c * L, L)]
            yg = plsc.load_gather(ybuf, [oyc])
            xg = plsc.load_gather(xbuf, [oxc])
            pv = iota + (c * L)
            vals = yg + xg
            plsc.store_scatter(idxa[slot], [jnp.minimum(pv, 103)], vals,
                               mask=pv < 104)
            plsc.store_scatter(idxb[slot], [jnp.clip(pv - 104, 0, 95)], vals,
                               mask=jnp.logical_and(pv >= 104, pv < G))
            wyv = plsc.load_gather(wyb, [oyc])
            wxv = plsc.load_gather(wxb, [oxc])
            wv[slot][pl.ds(c * L, L)] = wyv * wxv
        n = base + i
        nofs = n + 8 * (n // 8)
        for c in range(7):
            rv_ = iota + (c * L)
            plsc.store_scatter(idxo[slot], [jnp.minimum(rv_, 97)],
                               oz_v[pl.ds(c * L, L)] + nofs,
                               mask=rv_ < 98)

    def fire_gather(slot, gsem):
        pltpu.async_copy(table.at[idxa[slot]],
                         rows[slot].at[pl.ds(0, 104)], gsem)
        pltpu.async_copy(table.at[idxb[slot]],
                         rows[slot].at[pl.ds(104, 96)], gsem)

    def wait_gather(slot, gsem):
        pltpu.make_async_copy(table.at[idxa[slot]],
                              rows[slot].at[pl.ds(0, 104)], gsem).wait()
        pltpu.make_async_copy(table.at[idxb[slot]],
                              rows[slot].at[pl.ds(104, 96)], gsem).wait()

    def combine(i, slot):
        def s_body(s, carry):
            b = 4 * s
            wvv = wv[slot][pl.ds(b, L)]
            ws = [wvv[0], wvv[1], wvv[2], wvv[3]]
            for cb in range(CB // 2):
                sl = pl.ds(cb * L, L)
                r = rows[slot]
                acc_e = None
                acc_o = None
                for k in range(4):
                    e, o = plsc.unpack(
                        plsc.bitcast(r[b + k, sl], jnp.bfloat16),
                        format=plsc.PackFormat.INTERLEAVED)
                    if acc_e is None:
                        acc_e = e * ws[k]
                        acc_o = o * ws[k]
                    else:
                        acc_e = acc_e + e * ws[k]
                        acc_o = acc_o + o * ws[k]
                rr = jnp.full((L,), 2 * s + cb // 4, jnp.int32)
                plsc.store_scatter(out_t[slot], [rr, col_base[cb]], acc_e)
                plsc.store_scatter(out_t[slot], [rr, col_base[cb] + 1], acc_o)
            return carry
        lax.fori_loop(0, SS, s_body, 0)

    def fire_out(i, slot, osem):
        pltpu.async_copy(out_t[slot], out_hbm.at[idxo[slot]], osem)

    def wait_out(slot, osem):
        pltpu.make_async_copy(out_t[slot], out_hbm.at[idxo[slot]], osem).wait()

    setup(0, 0)
    fire_gather(0, gsemA)
    setup(1, 1)
    fire_gather(1, gsemB)

    def body(j, carry):
        i0 = 2 * j
        wait_gather(0, gsemA)

        @pl.when(j > 0)
        def _():
            wait_out(0, osemA)
        combine(i0, 0)
        fire_out(i0, 0, osemA)

        @pl.when(i0 + 2 < cnt)
        def _():
            setup(i0 + 2, 0)
            fire_gather(0, gsemA)

        wait_gather(1, gsemB)

        @pl.when(j > 0)
        def _():
            wait_out(1, osemB)
        combine(i0 + 1, 1)
        fire_out(i0 + 1, 1, osemB)

        @pl.when(i0 + 3 < cnt)
        def _():
            setup(i0 + 3, 1)
            fire_gather(1, gsemB)
        return carry

    lax.fori_loop(0, cnt // 2, body, 0)
    wait_out(0, osemA)
    wait_out(1, osemB)


def kernel(features, rois):
    feat = features[0]
    table = jnp.transpose(feat, (1, 2, 0)).reshape(H * W, C)
    table = jax.lax.bitcast_convert_type(
        table.astype(jnp.bfloat16).reshape(H * W, C // 2, 2), jnp.int32)
    n = rois.shape[0]
    oy = jnp.asarray(_OY)
    ox = jnp.asarray(_OX)
    oz = jnp.asarray(_OZ)
    out = _roialign_sc(table, rois.reshape(-1), oy, ox, oz)
    out = out.reshape(S, S, N // 8, 2, 8, 128)
    out = out.transpose(2, 4, 3, 5, 0, 1)
    return out.reshape(n, C, S, S)
